# baseline (speedup 1.0000x reference)
import functools

import jax
import jax.numpy as jnp
from jax import lax
from jax.experimental import pallas as pl
from jax.experimental.pallas import tpu as pltpu
from jax.experimental.pallas import tpu_sc as plsc

NC, NS = 2, 16
NW = NC * NS
NIMG, NTOK, D = 64, 1024, 64
IG = 8
DG = 4
IPW = NIMG // IG
DPW = D // DG


def kernel(indices, x_embed):
    idx2d = indices.astype(jnp.int32)
    tt_flat = x_embed.T.reshape(-1)

    mesh = plsc.VectorSubcoreMesh(
        core_axis_name="c", subcore_axis_name="s",
        num_cores=NC, num_subcores=NS)

    @functools.partial(
        pl.kernel,
        out_type=jax.ShapeDtypeStruct((NIMG, D, NTOK), jnp.float32),
        mesh=mesh,
        compiler_params=pltpu.CompilerParams(
            use_tc_tiling_on_sc=True, needs_layout_passes=False),
        scratch_types=[
            pltpu.VMEM((IPW, NTOK), jnp.int32),
            pltpu.VMEM((16384,), jnp.float32),
            pltpu.VMEM((2, DPW, NTOK), jnp.float32),
            pltpu.SemaphoreType.DMA,
        ],
    )
    def tgather(idx_hbm, tt_hbm, out_hbm, idx_v, tt_v, ob, sem):
        wid = lax.axis_index("s") * NC + lax.axis_index("c")
        ig = wid % IG
        dg = wid // IG
        pltpu.sync_copy(idx_hbm.at[pl.ds(ig * IPW, IPW), :], idx_v)
        pltpu.sync_copy(tt_hbm.at[pl.ds(dg * DPW * NTOK, DPW * NTOK)], tt_v)

        def do_image(im, buf):
            @plsc.parallel_loop(0, NTOK // 16, unroll=2)
            def body(g):
                iv = idx_v[im, pl.ds(g * 16, 16)]
                vals = [plsc.load_gather(tt_v, [iv + dd * NTOK])
                        for dd in range(DPW)]
                for dd in range(DPW):
                    ob[buf, dd, pl.ds(g * 16, 16)] = vals[dd]

        for im in range(IPW):
            buf = im % 2
            if im >= 2:
                pltpu.make_async_copy(
                    ob.at[buf],
                    out_hbm.at[ig * IPW + im - 2,
                               pl.ds(dg * DPW, DPW), :], sem).wait()
            do_image(im, buf)
            pltpu.async_copy(
                ob.at[buf],
                out_hbm.at[ig * IPW + im, pl.ds(dg * DPW, DPW), :], sem)
        for im in range(IPW - 2, IPW):
            buf = im % 2
            pltpu.make_async_copy(
                ob.at[buf],
                out_hbm.at[ig * IPW + im, pl.ds(dg * DPW, DPW), :], sem).wait()

    out = tgather(idx2d, tt_flat)
    return jnp.transpose(out, (0, 2, 1))


# speedup vs baseline: 1.0365x; 1.0365x over previous
import functools

import jax
import jax.numpy as jnp
from jax import lax
from jax.experimental import pallas as pl
from jax.experimental.pallas import tpu as pltpu
from jax.experimental.pallas import tpu_sc as plsc

NC, NS = 2, 16
NW = NC * NS
NIMG, NTOK, D = 64, 1024, 64
IG = 8
DG = 4
IPW = NIMG // IG
DPW = D // DG


def kernel(indices, x_embed):
    idx_flat = indices.reshape(-1).astype(jnp.int32)
    tt_flat = x_embed.T.reshape(-1)

    mesh = plsc.VectorSubcoreMesh(
        core_axis_name="c", subcore_axis_name="s",
        num_cores=NC, num_subcores=NS)

    @functools.partial(
        pl.kernel,
        out_type=jax.ShapeDtypeStruct((NIMG, D, NTOK), jnp.float32),
        mesh=mesh,
        compiler_params=pltpu.CompilerParams(
            use_tc_tiling_on_sc=True, needs_layout_passes=False),
        scratch_types=[
            pltpu.VMEM((IPW * NTOK,), jnp.int32),
            pltpu.VMEM((16384,), jnp.float32),
            pltpu.VMEM((2, DPW, NTOK), jnp.float32),
            pltpu.SemaphoreType.DMA,
            pltpu.SemaphoreType.DMA,
        ],
    )
    def tgather(idx_hbm, tt_hbm, out_hbm, idx_v, tt_v, ob, sem, psem):
        wid = lax.axis_index("s") * NC + lax.axis_index("c")
        ig = wid % IG
        dg = wid // IG
        pltpu.async_copy(idx_hbm.at[pl.ds(ig * IPW * NTOK, IPW * NTOK)],
                         idx_v, psem)
        pltpu.async_copy(tt_hbm.at[pl.ds(dg * DPW * NTOK, DPW * NTOK)],
                         tt_v, psem)
        pltpu.make_async_copy(
            idx_hbm.at[pl.ds(ig * IPW * NTOK, IPW * NTOK)], idx_v, psem).wait()
        pltpu.make_async_copy(
            tt_hbm.at[pl.ds(dg * DPW * NTOK, DPW * NTOK)], tt_v, psem).wait()

        def do_image(im, buf):
            @plsc.parallel_loop(0, NTOK // 16, unroll=2)
            def body(g):
                iv = idx_v[pl.ds(im * NTOK + g * 16, 16)]
                vals = [plsc.load_gather(tt_v, [iv + dd * NTOK])
                        for dd in range(DPW)]
                for dd in range(DPW):
                    ob[buf, dd, pl.ds(g * 16, 16)] = vals[dd]

        for im in range(IPW):
            buf = im % 2
            if im >= 2:
                pltpu.make_async_copy(
                    ob.at[buf],
                    out_hbm.at[ig * IPW + im - 2,
                               pl.ds(dg * DPW, DPW), :], sem).wait()
            do_image(im, buf)
            pltpu.async_copy(
                ob.at[buf],
                out_hbm.at[ig * IPW + im, pl.ds(dg * DPW, DPW), :], sem)
        for im in range(IPW - 2, IPW):
            buf = im % 2
            pltpu.make_async_copy(
                ob.at[buf],
                out_hbm.at[ig * IPW + im, pl.ds(dg * DPW, DPW), :], sem).wait()

    out = tgather(idx_flat, tt_flat)
    return jnp.transpose(out, (0, 2, 1))


# speedup vs baseline: 1.0407x; 1.0041x over previous
import functools

import jax
import jax.numpy as jnp
from jax import lax
from jax.experimental import pallas as pl
from jax.experimental.pallas import tpu as pltpu
from jax.experimental.pallas import tpu_sc as plsc

NC, NS = 2, 16
NW = NC * NS
NIMG, NTOK, D = 64, 1024, 64
IG = 8
DG = 4
IPW = NIMG // IG
DPW = D // DG


def kernel(indices, x_embed):
    idx_flat = indices.reshape(-1).astype(jnp.int32)
    tt_flat = x_embed.T.reshape(-1)

    mesh = plsc.VectorSubcoreMesh(
        core_axis_name="c", subcore_axis_name="s",
        num_cores=NC, num_subcores=NS)

    @functools.partial(
        pl.kernel,
        out_type=jax.ShapeDtypeStruct((NIMG, D, NTOK), jnp.float32),
        mesh=mesh,
        compiler_params=pltpu.CompilerParams(
            use_tc_tiling_on_sc=True, needs_layout_passes=False),
        scratch_types=[
            pltpu.VMEM((IPW * NTOK,), jnp.int32),
            pltpu.VMEM((16384,), jnp.float32),
            pltpu.VMEM((2, DPW, NTOK), jnp.float32),
            pltpu.SemaphoreType.DMA,
            pltpu.SemaphoreType.DMA,
            pltpu.SemaphoreType.DMA,
        ],
    )
    def tgather(idx_hbm, tt_hbm, out_hbm, idx_v, tt_v, ob, sem, psem, qsem):
        wid = lax.axis_index("s") * NC + lax.axis_index("c")
        ig = wid % IG
        dg = wid // IG
        pltpu.async_copy(idx_hbm.at[pl.ds(ig * IPW * NTOK, IPW * NTOK)],
                         idx_v, psem)
        pltpu.async_copy(tt_hbm.at[pl.ds(dg * DPW * NTOK, DPW * NTOK)],
                         tt_v, qsem)
        pltpu.make_async_copy(
            idx_hbm.at[pl.ds(ig * IPW * NTOK, IPW * NTOK)], idx_v, psem).wait()
        pltpu.make_async_copy(
            tt_hbm.at[pl.ds(dg * DPW * NTOK, DPW * NTOK)], tt_v, qsem).wait()

        def do_image(im, buf):
            @plsc.parallel_loop(0, NTOK // 16, unroll=2)
            def body(g):
                iv = idx_v[pl.ds(im * NTOK + g * 16, 16)]
                vals = [plsc.load_gather(tt_v, [iv + dd * NTOK])
                        for dd in range(DPW)]
                for dd in range(DPW):
                    ob[buf, dd, pl.ds(g * 16, 16)] = vals[dd]

        for im in range(IPW):
            buf = im % 2
            if im >= 2:
                pltpu.make_async_copy(
                    ob.at[buf],
                    out_hbm.at[ig * IPW + im - 2,
                               pl.ds(dg * DPW, DPW), :], sem).wait()
            do_image(im, buf)
            pltpu.async_copy(
                ob.at[buf],
                out_hbm.at[ig * IPW + im, pl.ds(dg * DPW, DPW), :], sem)
        for im in range(IPW - 2, IPW):
            buf = im % 2
            pltpu.make_async_copy(
                ob.at[buf],
                out_hbm.at[ig * IPW + im, pl.ds(dg * DPW, DPW), :], sem).wait()

    out = tgather(idx_flat, tt_flat)
    return jnp.transpose(out, (0, 2, 1))


# speedup vs baseline: 1.0451x; 1.0042x over previous
import functools

import jax
import jax.numpy as jnp
from jax import lax
from jax.experimental import pallas as pl
from jax.experimental.pallas import tpu as pltpu
from jax.experimental.pallas import tpu_sc as plsc

NC, NS = 2, 16
NW = NC * NS
NIMG, NTOK, D = 64, 1024, 64
IG = 8
DG = 4
IPW = NIMG // IG
DPW = D // DG


def kernel(indices, x_embed):
    idx_flat = indices.reshape(-1).astype(jnp.int32)
    tt_flat = x_embed.T.reshape(-1)

    mesh = plsc.VectorSubcoreMesh(
        core_axis_name="c", subcore_axis_name="s",
        num_cores=NC, num_subcores=NS)

    @functools.partial(
        pl.kernel,
        out_type=jax.ShapeDtypeStruct((NIMG, D, NTOK), jnp.float32),
        mesh=mesh,
        compiler_params=pltpu.CompilerParams(
            use_tc_tiling_on_sc=True, needs_layout_passes=False),
        scratch_types=[
            pltpu.VMEM((IPW * NTOK,), jnp.int32),
            pltpu.VMEM((DPW * NTOK,), jnp.float32),
            pltpu.VMEM((2, DPW, NTOK), jnp.float32),
            pltpu.SemaphoreType.DMA,
            pltpu.SemaphoreType.DMA,
            pltpu.SemaphoreType.DMA,
        ],
    )
    def tgather(idx_hbm, tt_hbm, out_hbm, idx_v, tt_v, ob, sem, psem, qsem):
        wid = lax.axis_index("s") * NC + lax.axis_index("c")
        ig = wid % IG
        dg = wid // IG
        pltpu.async_copy(idx_hbm.at[pl.ds(ig * IPW * NTOK, IPW * NTOK)],
                         idx_v, psem)
        pltpu.async_copy(tt_hbm.at[pl.ds(dg * DPW * NTOK, DPW * NTOK)],
                         tt_v, qsem)
        pltpu.make_async_copy(
            idx_hbm.at[pl.ds(ig * IPW * NTOK, IPW * NTOK)], idx_v, psem).wait()
        pltpu.make_async_copy(
            tt_hbm.at[pl.ds(dg * DPW * NTOK, DPW * NTOK)], tt_v, qsem).wait()

        def do_image(im, buf):
            @plsc.parallel_loop(0, NTOK // 16, unroll=2)
            def body(g):
                iv = idx_v[pl.ds(im * NTOK + g * 16, 16)]
                vals = [plsc.load_gather(tt_v, [iv + dd * NTOK])
                        for dd in range(DPW)]
                for dd in range(DPW):
                    ob[buf, dd, pl.ds(g * 16, 16)] = vals[dd]

        for im in range(IPW):
            buf = im % 2
            if im >= 2:
                pltpu.make_async_copy(
                    ob.at[buf],
                    out_hbm.at[ig * IPW + im - 2,
                               pl.ds(dg * DPW, DPW), :], sem).wait()
            do_image(im, buf)
            pltpu.async_copy(
                ob.at[buf],
                out_hbm.at[ig * IPW + im, pl.ds(dg * DPW, DPW), :], sem)
        for im in range(IPW - 2, IPW):
            buf = im % 2
            pltpu.make_async_copy(
                ob.at[buf],
                out_hbm.at[ig * IPW + im, pl.ds(dg * DPW, DPW), :], sem).wait()

    out = tgather(idx_flat, tt_flat)
    return jnp.transpose(out, (0, 2, 1))
